# baseline (speedup 1.0000x reference)
import functools

import jax
import jax.numpy as jnp
from jax import lax
from jax.experimental import pallas as pl
from jax.experimental.pallas import tpu as pltpu
from jax.experimental.pallas import tpu_sc as plsc

_BLOCK_T = 2048


def _add_kernel(x_ref, p_ref, o_ref):
    o_ref[...] = x_ref[...] + p_ref[...]


def _tc_kernel(x, pos_embedding):
    B, T, D = x.shape
    pos = pos_embedding[:T]
    bt = min(_BLOCK_T, T)
    grid = (T // bt, B)
    return pl.pallas_call(
        _add_kernel,
        grid=grid,
        in_specs=[
            pl.BlockSpec((1, bt, D), lambda t, b: (b, t, 0)),
            pl.BlockSpec((bt, D), lambda t, b: (t, 0)),
        ],
        out_specs=pl.BlockSpec((1, bt, D), lambda t, b: (b, t, 0)),
        out_shape=jax.ShapeDtypeStruct((B, T, D), x.dtype),
    )(x, pos)


_SC_CHUNK = 16384
_SC_LANES = 16


def _sc_kernel(x, pos_embedding):
    B, T, D = x.shape
    pos = pos_embedding[:T]
    total = B * T * D
    pos_words = T * D
    nc, ns = 2, 16
    nw = nc * ns
    wpw = total // nw
    workers_per_batch = pos_words // wpw
    nchunk = wpw // _SC_CHUNK

    mesh = plsc.VectorSubcoreMesh(
        core_axis_name="c", subcore_axis_name="s", num_cores=nc, num_subcores=ns
    )

    @functools.partial(
        pl.kernel,
        mesh=mesh,
        out_type=jax.ShapeDtypeStruct((total,), jnp.float32),
        scratch_types=[
            pltpu.VMEM((_SC_CHUNK,), jnp.float32),
            pltpu.VMEM((_SC_CHUNK,), jnp.float32),
        ],
    )
    def sc_add(x_hbm, pos_hbm, out_hbm, xbuf, pbuf):
        c = lax.axis_index("c")
        s = lax.axis_index("s")
        wid = s * nc + c
        base = wid * wpw
        pbase = lax.rem(wid, workers_per_batch) * wpw

        def chunk_body(i, carry):
            off = pl.multiple_of(base + i * _SC_CHUNK, _SC_CHUNK)
            poff = pl.multiple_of(pbase + i * _SC_CHUNK, _SC_CHUNK)
            pltpu.sync_copy(x_hbm.at[pl.ds(off, _SC_CHUNK)], xbuf)
            pltpu.sync_copy(pos_hbm.at[pl.ds(poff, _SC_CHUNK)], pbuf)

            def add_body(j, carry2):
                o = pl.multiple_of(j * _SC_LANES, _SC_LANES)
                xbuf[pl.ds(o, _SC_LANES)] = (
                    xbuf[pl.ds(o, _SC_LANES)] + pbuf[pl.ds(o, _SC_LANES)]
                )
                return carry2

            lax.fori_loop(0, _SC_CHUNK // _SC_LANES, add_body, 0)
            pltpu.sync_copy(xbuf, out_hbm.at[pl.ds(off, _SC_CHUNK)])
            return carry

        lax.fori_loop(0, nchunk, chunk_body, 0)

    out = sc_add(x.reshape(total), pos.reshape(pos_words))
    return out.reshape(B, T, D)


kernel = _sc_kernel


# speedup vs baseline: 1.6544x; 1.6544x over previous
import functools

import jax
import jax.numpy as jnp
from jax import lax
from jax.experimental import pallas as pl
from jax.experimental.pallas import tpu as pltpu
from jax.experimental.pallas import tpu_sc as plsc

_BLOCK_T = 2048


def _add_kernel(x_ref, p_ref, o_ref):
    o_ref[...] = x_ref[...] + p_ref[...]


def _tc_kernel(x, pos_embedding):
    B, T, D = x.shape
    pos = pos_embedding[:T]
    bt = min(_BLOCK_T, T)
    grid = (T // bt, B)
    return pl.pallas_call(
        _add_kernel,
        grid=grid,
        in_specs=[
            pl.BlockSpec((1, bt, D), lambda t, b: (b, t, 0)),
            pl.BlockSpec((bt, D), lambda t, b: (t, 0)),
        ],
        out_specs=pl.BlockSpec((1, bt, D), lambda t, b: (b, t, 0)),
        out_shape=jax.ShapeDtypeStruct((B, T, D), x.dtype),
    )(x, pos)


_SC_CHUNK = 16384
_SC_LANES = 16
_SC_UNROLL = 8


def _sc_kernel_v2(x, pos_embedding):
    B, T, D = x.shape
    pos = pos_embedding[:T]
    total = B * T * D
    pos_words = T * D
    nc, ns = 2, 16
    nw = nc * ns
    wpw = total // nw
    workers_per_batch = pos_words // wpw
    nchunk = wpw // _SC_CHUNK
    ngroup = nchunk // 2
    ch = _SC_CHUNK

    mesh = plsc.VectorSubcoreMesh(
        core_axis_name="c", subcore_axis_name="s", num_cores=nc, num_subcores=ns
    )

    @functools.partial(
        pl.kernel,
        mesh=mesh,
        out_type=jax.ShapeDtypeStruct((total,), jnp.float32),
        scratch_types=[
            pltpu.VMEM((2, ch), jnp.float32),
            pltpu.VMEM((2, ch), jnp.float32),
            pltpu.VMEM((2, ch), jnp.float32),
            pltpu.SemaphoreType.DMA,
            pltpu.SemaphoreType.DMA,
            pltpu.SemaphoreType.DMA,
            pltpu.SemaphoreType.DMA,
            pltpu.SemaphoreType.DMA,
            pltpu.SemaphoreType.DMA,
        ],
    )
    def sc_add(x_hbm, pos_hbm, out_hbm, xbuf, pbuf, obuf,
               xs0, xs1, ps0, ps1, os0, os1):
        c = lax.axis_index("c")
        s = lax.axis_index("s")
        wid = s * nc + c
        base = wid * wpw
        pbase = lax.rem(wid, workers_per_batch) * wpw
        xsems = (xs0, xs1)
        psems = (ps0, ps1)
        osems = (os0, os1)

        def start_in(k, b):
            off = pl.multiple_of(base + k * ch, ch)
            poff = pl.multiple_of(pbase + k * ch, ch)
            pltpu.make_async_copy(
                x_hbm.at[pl.ds(off, ch)], xbuf.at[b], xsems[b]
            ).start()
            pltpu.make_async_copy(
                pos_hbm.at[pl.ds(poff, ch)], pbuf.at[b], psems[b]
            ).start()

        def wait_in(k, b):
            off = pl.multiple_of(base + k * ch, ch)
            poff = pl.multiple_of(pbase + k * ch, ch)
            pltpu.make_async_copy(
                x_hbm.at[pl.ds(off, ch)], xbuf.at[b], xsems[b]
            ).wait()
            pltpu.make_async_copy(
                pos_hbm.at[pl.ds(poff, ch)], pbuf.at[b], psems[b]
            ).wait()

        def start_out(k, b):
            off = pl.multiple_of(base + k * ch, ch)
            pltpu.make_async_copy(
                obuf.at[b], out_hbm.at[pl.ds(off, ch)], osems[b]
            ).start()

        def wait_out(k, b):
            off = pl.multiple_of(base + k * ch, ch)
            pltpu.make_async_copy(
                obuf.at[b], out_hbm.at[pl.ds(off, ch)], osems[b]
            ).wait()

        def add_chunk(b):
            def add_body(j, carry):
                jo = j * (_SC_LANES * _SC_UNROLL)
                for u in range(_SC_UNROLL):
                    o = pl.multiple_of(jo + u * _SC_LANES, _SC_LANES)
                    obuf[b, pl.ds(o, _SC_LANES)] = (
                        xbuf[b, pl.ds(o, _SC_LANES)]
                        + pbuf[b, pl.ds(o, _SC_LANES)]
                    )
                return carry

            lax.fori_loop(0, ch // (_SC_LANES * _SC_UNROLL), add_body, 0)

        start_in(0, 0)
        start_in(1, 1)
        for b in range(2):
            wait_in(b, b)
            add_chunk(b)
            start_out(b, b)
            start_in(b + 2, b)

        def group_body(g, carry):
            for b in range(2):
                k = 2 * g + b
                wait_in(k, b)
                wait_out(k - 2, b)
                add_chunk(b)
                start_out(k, b)
                start_in(k + 2, b)
            return carry

        lax.fori_loop(1, ngroup - 1, group_body, 0)

        for b in range(2):
            k = 2 * (ngroup - 1) + b
            wait_in(k, b)
            wait_out(k - 2, b)
            add_chunk(b)
            start_out(k, b)
        for b in range(2):
            wait_out(2 * (ngroup - 1) + b, b)

    out = sc_add(x.reshape(total), pos.reshape(pos_words))
    return out.reshape(B, T, D)


def _sc_kernel(x, pos_embedding):
    B, T, D = x.shape
    pos = pos_embedding[:T]
    total = B * T * D
    pos_words = T * D
    nc, ns = 2, 16
    nw = nc * ns
    wpw = total // nw
    workers_per_batch = pos_words // wpw
    nchunk = wpw // _SC_CHUNK

    mesh = plsc.VectorSubcoreMesh(
        core_axis_name="c", subcore_axis_name="s", num_cores=nc, num_subcores=ns
    )

    @functools.partial(
        pl.kernel,
        mesh=mesh,
        out_type=jax.ShapeDtypeStruct((total,), jnp.float32),
        scratch_types=[
            pltpu.VMEM((_SC_CHUNK,), jnp.float32),
            pltpu.VMEM((_SC_CHUNK,), jnp.float32),
        ],
    )
    def sc_add(x_hbm, pos_hbm, out_hbm, xbuf, pbuf):
        c = lax.axis_index("c")
        s = lax.axis_index("s")
        wid = s * nc + c
        base = wid * wpw
        pbase = lax.rem(wid, workers_per_batch) * wpw

        def chunk_body(i, carry):
            off = pl.multiple_of(base + i * _SC_CHUNK, _SC_CHUNK)
            poff = pl.multiple_of(pbase + i * _SC_CHUNK, _SC_CHUNK)
            pltpu.sync_copy(x_hbm.at[pl.ds(off, _SC_CHUNK)], xbuf)
            pltpu.sync_copy(pos_hbm.at[pl.ds(poff, _SC_CHUNK)], pbuf)

            def add_body(j, carry2):
                o = pl.multiple_of(j * _SC_LANES, _SC_LANES)
                xbuf[pl.ds(o, _SC_LANES)] = (
                    xbuf[pl.ds(o, _SC_LANES)] + pbuf[pl.ds(o, _SC_LANES)]
                )
                return carry2

            lax.fori_loop(0, _SC_CHUNK // _SC_LANES, add_body, 0)
            pltpu.sync_copy(xbuf, out_hbm.at[pl.ds(off, _SC_CHUNK)])
            return carry

        lax.fori_loop(0, nchunk, chunk_body, 0)

    out = sc_add(x.reshape(total), pos.reshape(pos_words))
    return out.reshape(B, T, D)


kernel = _sc_kernel_v2


# speedup vs baseline: 8.5707x; 5.1807x over previous
import jax
import jax.numpy as jnp
from jax.experimental import pallas as pl

_BLOCK_T = 2048


def _add_kernel(x_ref, p_ref, o_ref):
    o_ref[...] = x_ref[...] + p_ref[...]


def kernel(x, pos_embedding):
    B, T, D = x.shape
    pos = pos_embedding[:T]
    bt = min(_BLOCK_T, T)
    grid = (T // bt, B)
    return pl.pallas_call(
        _add_kernel,
        grid=grid,
        in_specs=[
            pl.BlockSpec((1, bt, D), lambda t, b: (b, t, 0)),
            pl.BlockSpec((bt, D), lambda t, b: (t, 0)),
        ],
        out_specs=pl.BlockSpec((1, bt, D), lambda t, b: (b, t, 0)),
        out_shape=jax.ShapeDtypeStruct((B, T, D), x.dtype),
    )(x, pos)
